# baseline (speedup 1.0000x reference)
import functools

import jax
import jax.numpy as jnp
from jax import lax
from jax.experimental import pallas as pl
from jax.experimental.pallas import tpu as pltpu
from jax.experimental.pallas import tpu_sc as plsc

_N = 8192
_D = 128
_Q = 65536
_NC = 2
_NS = 16
_DG = _D // _NS
_QPH = _Q // _NC
_QSH = _QPH // _NS
_CQ = 512
_L = 16

_GD = lax.GatherDimensionNumbers(
    offset_dims=(), collapsed_slice_dims=(0,), start_index_map=(0,))


def _weight_t_body(emb_ref, bias_ref, sw_ref, o_ref):
    o_ref[...] = jnp.transpose((emb_ref[...] + bias_ref[...]) * sw_ref[...])


def _make_weighted_t(emb, bias, sw):
    return pl.pallas_call(
        _weight_t_body,
        out_shape=jax.ShapeDtypeStruct((_D, _N), jnp.float32),
    )(emb, bias, sw.reshape(1, _D))


def _sc_body(wt_hbm, t_hbm, tens_hbm, curv_hbm, out_hbm,
             tbl_v, tens_v, curv_v, t_own, all5,
             sh_all, bufIA, bufIB,
             outA, outB, semIA, semIB, semOA, semOB):
    h = lax.axis_index("c")
    s = lax.axis_index("s")
    qhalf = h * _QPH

    pltpu.sync_copy(wt_hbm.at[pl.ds(s * _DG, _DG)], tbl_v)
    pltpu.sync_copy(tens_hbm, tens_v)
    pltpu.sync_copy(curv_hbm, curv_v)
    pltpu.sync_copy(t_hbm.at[pl.ds(qhalf + s * _QSH, _QSH)], t_own)

    lanes = lax.iota(jnp.int32, _L)
    pos5 = lanes * 5

    def pre(k, carry):
        sl = pl.ds(k * _L, _L)
        tv = t_own[sl]
        ts = tv * float(_N - 1)
        seg = jnp.clip(ts.astype(jnp.int32), 0, _N - 2)
        tl = ts - seg.astype(jnp.float32)
        tens = plsc.load_gather(tens_v, [seg])
        sig = 1.0 / (1.0 + jnp.exp(-tens))
        c1 = plsc.load_gather(curv_v, [seg])
        c2 = plsc.load_gather(curv_v, [seg + 1])
        t2 = tl * tl
        t3 = t2 * tl
        v0 = (-0.5 * t3 + t2 - 0.5 * tl) * sig
        v1 = (1.5 * t3 - 2.5 * t2 + 1.0) * c1
        v2 = (-1.5 * t3 + 2.0 * t2 + 0.5 * tl) * c2
        v3 = (0.5 * t3 - 0.5 * t2) * sig
        rcp = 1.0 / (v0 + v1 + v2 + v3)
        base = k * (5 * _L)
        all5[pl.ds(base, _L)] = plsc.bitcast(seg, jnp.float32)
        all5[pl.ds(base + _L, _L)] = v0 * rcp
        all5[pl.ds(base + 2 * _L, _L)] = v1 * rcp
        all5[pl.ds(base + 3 * _L, _L)] = v2 * rcp
        all5[pl.ds(base + 4 * _L, _L)] = v3 * rcp
        return carry

    lax.fori_loop(0, _QSH // _L, pre, 0)
    pltpu.sync_copy(all5, sh_all.at[s])
    plsc.subcore_barrier()

    def blend(buf, out_v):
        def bg(g, carry):
            base = g * (5 * _L)
            segb = buf[pl.ds(base, _L)]
            w0 = buf[pl.ds(base + _L, _L)]
            w1 = buf[pl.ds(base + 2 * _L, _L)]
            w2 = buf[pl.ds(base + 3 * _L, _L)]
            w3 = buf[pl.ds(base + 4 * _L, _L)]
            seg = plsc.bitcast(segb, jnp.int32)
            r0 = jnp.maximum(seg - 1, 0)
            r2 = seg + 1
            r3 = jnp.minimum(seg + 2, _N - 1)
            qrow = g * _L + lanes
            for half in range(2):
                loaded = []
                for d in range(half * 4, half * 4 + 4):
                    cd = jnp.full((_L,), d, jnp.int32)
                    loaded.append((cd,
                                   plsc.load_gather(tbl_v, [cd, r0]),
                                   plsc.load_gather(tbl_v, [cd, seg]),
                                   plsc.load_gather(tbl_v, [cd, r2]),
                                   plsc.load_gather(tbl_v, [cd, r3])))
                for cd, v0, v1, v2, v3 in loaded:
                    acc = (v0 * w0 + v1 * w1) + (v2 * w2 + v3 * w3)
                    plsc.store_scatter(out_v, [qrow, cd], acc)
            return carry

        lax.fori_loop(0, _CQ // _L, bg, 0, unroll=4)

    def fire_in(b, j, buf, sem):
        pltpu.async_copy(sh_all.at[b, pl.ds(j * (5 * _CQ), 5 * _CQ)], buf, sem)

    def drain_in(buf, sem):
        pltpu.make_async_copy(t_hbm.at[pl.ds(0, 5 * _CQ)], buf, sem).wait()

    def store(b, j, out_v, sem):
        qg = qhalf + b * _QSH + j * _CQ
        pltpu.async_copy(out_v, out_hbm.at[pl.ds(qg, _CQ), pl.ds(s * _DG, _DG)],
                         sem)

    def drain_store(out_v, sem):
        pltpu.make_async_copy(
            out_v, out_hbm.at[pl.ds(qhalf, _CQ), pl.ds(s * _DG, _DG)],
            sem).wait()

    fire_in(0, 0, bufIA, semIA)

    def body(b, carry):
        fire_in(b, 1, bufIB, semIB)
        drain_in(bufIA, semIA)

        @pl.when(b > 0)
        def _():
            drain_store(outA, semOA)

        blend(bufIA, outA)
        store(b, 0, outA, semOA)

        fire_in(b, 2, bufIA, semIA)
        drain_in(bufIB, semIB)

        @pl.when(b > 0)
        def _():
            drain_store(outB, semOB)

        blend(bufIB, outB)
        store(b, 1, outB, semOB)

        fire_in(b, 3, bufIB, semIB)
        drain_in(bufIA, semIA)
        drain_store(outA, semOA)
        blend(bufIA, outA)
        store(b, 2, outA, semOA)

        fire_in(jnp.minimum(b + 1, _NS - 1), 0, bufIA, semIA)
        drain_in(bufIB, semIB)
        drain_store(outB, semOB)
        blend(bufIB, outB)
        store(b, 3, outB, semOB)
        return carry

    lax.fori_loop(0, _NS, body, 0)
    drain_in(bufIA, semIA)
    drain_store(outA, semOA)
    drain_store(outB, semOB)


@functools.partial(
    pl.kernel,
    out_type=jax.ShapeDtypeStruct((_Q, _D), jnp.float32),
    mesh=plsc.VectorSubcoreMesh(core_axis_name="c", subcore_axis_name="s"),
    scratch_types=[
        pltpu.VMEM((_DG, _N), jnp.float32),
        pltpu.VMEM((_N,), jnp.float32),
        pltpu.VMEM((_N,), jnp.float32),
        pltpu.VMEM((_QSH,), jnp.float32),
        pltpu.VMEM((5 * _QSH,), jnp.float32),
        pltpu.VMEM_SHARED((_NS, 5 * _QSH), jnp.float32),
        pltpu.VMEM((5 * _CQ,), jnp.float32),
        pltpu.VMEM((5 * _CQ,), jnp.float32),
        pltpu.VMEM((_CQ, _DG), jnp.float32),
        pltpu.VMEM((_CQ, _DG), jnp.float32),
        pltpu.SemaphoreType.DMA,
        pltpu.SemaphoreType.DMA,
        pltpu.SemaphoreType.DMA,
        pltpu.SemaphoreType.DMA,
    ],
    compiler_params=pltpu.CompilerParams(
        needs_layout_passes=False, use_tc_tiling_on_sc=False),
)
def _sc_spline(wt_hbm, t_hbm, tens_hbm, curv_hbm, out_hbm, *scratch):
    _sc_body(wt_hbm, t_hbm, tens_hbm, curv_hbm, out_hbm, *scratch)


def kernel(word_embeddings, t_query, tension_params, semantic_weights,
           word_biases, curvature_controls):
    weighted = _make_weighted_t(word_embeddings, word_biases, semantic_weights)
    tens_pad = jnp.pad(tension_params, (0, 1))
    return _sc_spline(weighted, t_query, tens_pad, curvature_controls)


# speedup vs baseline: 1.0053x; 1.0053x over previous
import functools

import jax
import jax.numpy as jnp
from jax import lax
from jax.experimental import pallas as pl
from jax.experimental.pallas import tpu as pltpu
from jax.experimental.pallas import tpu_sc as plsc

_N = 8192
_D = 128
_Q = 65536
_NC = 2
_NS = 16
_DG = _D // _NS
_QPH = _Q // _NC
_QSH = _QPH // _NS
_CQ = 512
_L = 16

def _weight_t_body(emb_ref, bias_ref, sw_ref, o_ref):
    o_ref[...] = jnp.transpose((emb_ref[...] + bias_ref[...]) * sw_ref[...])


def _make_weighted_t(emb, bias, sw):
    return pl.pallas_call(
        _weight_t_body,
        out_shape=jax.ShapeDtypeStruct((_D, _N), jnp.float32),
    )(emb, bias, sw.reshape(1, _D))


def _sc_body(wt_hbm, t_hbm, tens_hbm, curv_hbm, out_hbm,
             tbl_v, tens_v, curv_v, t_own, all5,
             sh_all, bufIA, bufIB,
             outA, outB, semIA, semIB, semOA, semOB):
    h = lax.axis_index("c")
    s = lax.axis_index("s")
    qhalf = h * _QPH

    pltpu.sync_copy(wt_hbm.at[pl.ds(s * _DG, _DG)], tbl_v)
    pltpu.sync_copy(tens_hbm, tens_v)
    pltpu.sync_copy(curv_hbm, curv_v)
    pltpu.sync_copy(t_hbm.at[pl.ds(qhalf + s * _QSH, _QSH)], t_own)

    lanes = lax.iota(jnp.int32, _L)

    def pre(k, carry):
        sl = pl.ds(k * _L, _L)
        tv = t_own[sl]
        ts = tv * float(_N - 1)
        seg = jnp.clip(ts.astype(jnp.int32), 0, _N - 2)
        tl = ts - seg.astype(jnp.float32)
        tens = plsc.load_gather(tens_v, [seg])
        sig = 1.0 / (1.0 + jnp.exp(-tens))
        c1 = plsc.load_gather(curv_v, [seg])
        c2 = plsc.load_gather(curv_v, [seg + 1])
        t2 = tl * tl
        t3 = t2 * tl
        v0 = (-0.5 * t3 + t2 - 0.5 * tl) * sig
        v1 = (1.5 * t3 - 2.5 * t2 + 1.0) * c1
        v2 = (-1.5 * t3 + 2.0 * t2 + 0.5 * tl) * c2
        v3 = (0.5 * t3 - 0.5 * t2) * sig
        rcp = 1.0 / (v0 + v1 + v2 + v3)
        base = k * (5 * _L)
        all5[pl.ds(base, _L)] = plsc.bitcast(seg, jnp.float32)
        all5[pl.ds(base + _L, _L)] = v0 * rcp
        all5[pl.ds(base + 2 * _L, _L)] = v1 * rcp
        all5[pl.ds(base + 3 * _L, _L)] = v2 * rcp
        all5[pl.ds(base + 4 * _L, _L)] = v3 * rcp
        return carry

    lax.fori_loop(0, _QSH // _L, pre, 0)
    pltpu.sync_copy(all5, sh_all.at[s])
    plsc.subcore_barrier()

    def blend(buf, out_v):
        def bg(g, carry):
            base = g * (5 * _L)
            segb = buf[pl.ds(base, _L)]
            w0 = buf[pl.ds(base + _L, _L)]
            w1 = buf[pl.ds(base + 2 * _L, _L)]
            w2 = buf[pl.ds(base + 3 * _L, _L)]
            w3 = buf[pl.ds(base + 4 * _L, _L)]
            seg = plsc.bitcast(segb, jnp.int32)
            r0 = jnp.maximum(seg - 1, 0)
            r2 = seg + 1
            r3 = jnp.minimum(seg + 2, _N - 1)
            qrow = g * _L + lanes
            for half in range(2):
                loaded = []
                for d in range(half * 4, half * 4 + 4):
                    cd = jnp.full((_L,), d, jnp.int32)
                    loaded.append((cd,
                                   plsc.load_gather(tbl_v, [cd, r0]),
                                   plsc.load_gather(tbl_v, [cd, seg]),
                                   plsc.load_gather(tbl_v, [cd, r2]),
                                   plsc.load_gather(tbl_v, [cd, r3])))
                for cd, v0, v1, v2, v3 in loaded:
                    acc = (v0 * w0 + v1 * w1) + (v2 * w2 + v3 * w3)
                    plsc.store_scatter(out_v, [qrow, cd], acc)
            return carry

        lax.fori_loop(0, _CQ // _L, bg, 0, unroll=2)

    def fire_in(b, j, buf, sem):
        pltpu.async_copy(sh_all.at[b, pl.ds(j * (5 * _CQ), 5 * _CQ)], buf, sem)

    def drain_in(buf, sem):
        pltpu.make_async_copy(t_hbm.at[pl.ds(0, 5 * _CQ)], buf, sem).wait()

    def store(b, j, out_v, sem):
        qg = qhalf + b * _QSH + j * _CQ
        pltpu.async_copy(out_v, out_hbm.at[pl.ds(qg, _CQ), pl.ds(s * _DG, _DG)],
                         sem)

    def drain_store(out_v, sem):
        pltpu.make_async_copy(
            out_v, out_hbm.at[pl.ds(qhalf, _CQ), pl.ds(s * _DG, _DG)],
            sem).wait()

    fire_in(0, 0, bufIA, semIA)

    def body(b, carry):
        fire_in(b, 1, bufIB, semIB)
        drain_in(bufIA, semIA)

        @pl.when(b > 0)
        def _():
            drain_store(outA, semOA)

        blend(bufIA, outA)
        store(b, 0, outA, semOA)

        fire_in(b, 2, bufIA, semIA)
        drain_in(bufIB, semIB)

        @pl.when(b > 0)
        def _():
            drain_store(outB, semOB)

        blend(bufIB, outB)
        store(b, 1, outB, semOB)

        fire_in(b, 3, bufIB, semIB)
        drain_in(bufIA, semIA)
        drain_store(outA, semOA)
        blend(bufIA, outA)
        store(b, 2, outA, semOA)

        fire_in(jnp.minimum(b + 1, _NS - 1), 0, bufIA, semIA)
        drain_in(bufIB, semIB)
        drain_store(outB, semOB)
        blend(bufIB, outB)
        store(b, 3, outB, semOB)
        return carry

    lax.fori_loop(0, _NS, body, 0)
    drain_in(bufIA, semIA)
    drain_store(outA, semOA)
    drain_store(outB, semOB)


@functools.partial(
    pl.kernel,
    out_type=jax.ShapeDtypeStruct((_Q, _D), jnp.float32),
    mesh=plsc.VectorSubcoreMesh(core_axis_name="c", subcore_axis_name="s"),
    scratch_types=[
        pltpu.VMEM((_DG, _N), jnp.float32),
        pltpu.VMEM((_N,), jnp.float32),
        pltpu.VMEM((_N,), jnp.float32),
        pltpu.VMEM((_QSH,), jnp.float32),
        pltpu.VMEM((5 * _QSH,), jnp.float32),
        pltpu.VMEM_SHARED((_NS, 5 * _QSH), jnp.float32),
        pltpu.VMEM((5 * _CQ,), jnp.float32),
        pltpu.VMEM((5 * _CQ,), jnp.float32),
        pltpu.VMEM((_CQ, _DG), jnp.float32),
        pltpu.VMEM((_CQ, _DG), jnp.float32),
        pltpu.SemaphoreType.DMA,
        pltpu.SemaphoreType.DMA,
        pltpu.SemaphoreType.DMA,
        pltpu.SemaphoreType.DMA,
    ],
    compiler_params=pltpu.CompilerParams(
        needs_layout_passes=False, use_tc_tiling_on_sc=False),
)
def _sc_spline(wt_hbm, t_hbm, tens_hbm, curv_hbm, out_hbm, *scratch):
    _sc_body(wt_hbm, t_hbm, tens_hbm, curv_hbm, out_hbm, *scratch)


def kernel(word_embeddings, t_query, tension_params, semantic_weights,
           word_biases, curvature_controls):
    weighted = _make_weighted_t(word_embeddings, word_biases, semantic_weights)
    tens_pad = jnp.pad(tension_params, (0, 1))
    return _sc_spline(weighted, t_query, tens_pad, curvature_controls)
